# baseline (speedup 1.0000x reference)
import functools

import jax
import jax.numpy as jnp
from jax import lax
from jax.experimental import pallas as pl
from jax.experimental.pallas import tpu as pltpu
from jax.experimental.pallas import tpu_sc as plsc

_VOCAB_PAD = 30720
_BLK = 2048
_SEQ = 200
_BATCH = 4096
_OUT = 2
_NC, _NS, _L = 2, 16, 16
_NW = _NC * _NS
_BPW = _BATCH // _NW
_G = _BPW // _L


def _project_body(e_ref, w_ref, b_ref, t_ref):
    t_ref[...] = (
        jnp.dot(e_ref[...], w_ref[...], preferred_element_type=jnp.float32)
        + b_ref[...]
    )


def _project(emb_table, W, b):
    d = emb_table.shape[1]
    return pl.pallas_call(
        _project_body,
        grid=(_VOCAB_PAD // _BLK,),
        in_specs=[
            pl.BlockSpec((_BLK, d), lambda i: (i, 0)),
            pl.BlockSpec((d, _OUT), lambda i: (0, 0)),
            pl.BlockSpec((1, _OUT), lambda i: (0, 0)),
        ],
        out_specs=pl.BlockSpec((_BLK, _OUT), lambda i: (i, 0)),
        out_shape=jax.ShapeDtypeStruct((_VOCAB_PAD, _OUT), jnp.float32),
    )(emb_table, W, b.reshape(1, _OUT))


def _xpose_body(x_ref, o_ref):
    o_ref[...] = x_ref[...].T[None]


def _xpose(x):
    return pl.pallas_call(
        _xpose_body,
        grid=(_NW,),
        in_specs=[pl.BlockSpec((_BPW, _SEQ), lambda i: (i, 0))],
        out_specs=pl.BlockSpec((1, _SEQ, _BPW), lambda i: (i, 0, 0)),
        out_shape=jax.ShapeDtypeStruct((_NW, _SEQ, _BPW), jnp.int32),
    )(x)


def _sc_body(t_hbm, x_hbm, out_hbm, t_v, x_v, o_v):
    wid = lax.axis_index("s") * _NC + lax.axis_index("c")
    pltpu.sync_copy(t_hbm, t_v)
    pltpu.sync_copy(x_hbm.at[wid], x_v)
    inv_l = jnp.float32(1.0 / _SEQ)
    z = jnp.zeros((_L,), jnp.float32)
    for g in range(_G):
        @plsc.parallel_loop(0, _SEQ, step=2, unroll=4, carry=(z, z, z, z))
        def body(l, accs, _g=g):
            a0, a1, b0, b1 = accs
            idx2 = x_v[l, pl.ds(_g * _L, _L)] * 2
            jdx2 = x_v[l + 1, pl.ds(_g * _L, _L)] * 2
            a0 = a0 + plsc.load_gather(t_v, [idx2])
            a1 = a1 + plsc.load_gather(t_v, [idx2 + 1])
            b0 = b0 + plsc.load_gather(t_v, [jdx2])
            b1 = b1 + plsc.load_gather(t_v, [jdx2 + 1])
            return a0, a1, b0, b1

        a0, a1, b0, b1 = body
        o_v[0, pl.ds(g * _L, _L)] = (a0 + b0) * inv_l
        o_v[1, pl.ds(g * _L, _L)] = (a1 + b1) * inv_l
    pltpu.sync_copy(o_v, out_hbm.at[wid])


_sc_pool = functools.partial(
    pl.kernel,
    out_type=jax.ShapeDtypeStruct((_NW, _OUT, _BPW), jnp.float32),
    mesh=plsc.VectorSubcoreMesh(
        core_axis_name="c", subcore_axis_name="s",
        num_cores=_NC, num_subcores=_NS,
    ),
    scratch_types=[
        pltpu.VMEM((_VOCAB_PAD * _OUT,), jnp.float32),
        pltpu.VMEM((_SEQ, _BPW), jnp.int32),
        pltpu.VMEM((_OUT, _BPW), jnp.float32),
    ],
    compiler_params=pltpu.CompilerParams(needs_layout_passes=False),
)(_sc_body)


def kernel(x, attention_mask, emb_table, W, b):
    t = _project(emb_table, W, b).reshape(-1)
    x2 = _xpose(x)
    out = _sc_pool(t, x2)
    return out.transpose(0, 2, 1).reshape(_BATCH, _OUT)


# speedup vs baseline: 1.0572x; 1.0572x over previous
import functools

import jax
import jax.numpy as jnp
from jax import lax
from jax.experimental import pallas as pl
from jax.experimental.pallas import tpu as pltpu
from jax.experimental.pallas import tpu_sc as plsc

_VOCAB_PAD = 30720
_BLK = 2048
_SEQ = 200
_BATCH = 4096
_OUT = 2
_NC, _NS, _L = 2, 16, 16
_NW = _NC * _NS
_BPW = _BATCH // _NW
_G = _BPW // _L


def _project_body(e_ref, w_ref, b_ref, t_ref):
    t_ref[...] = (
        jnp.dot(e_ref[...], w_ref[...], preferred_element_type=jnp.float32)
        + b_ref[...]
    )


def _project(emb_table, W, b):
    d = emb_table.shape[1]
    return pl.pallas_call(
        _project_body,
        grid=(_VOCAB_PAD // _BLK,),
        in_specs=[
            pl.BlockSpec((_BLK, d), lambda i: (i, 0)),
            pl.BlockSpec((d, _OUT), lambda i: (0, 0)),
            pl.BlockSpec((1, _OUT), lambda i: (0, 0)),
        ],
        out_specs=pl.BlockSpec((_BLK, _OUT), lambda i: (i, 0)),
        out_shape=jax.ShapeDtypeStruct((_VOCAB_PAD, _OUT), jnp.float32),
    )(emb_table, W, b.reshape(1, _OUT))


_SEQP = 256


def _sc_body(t_hbm, x_hbm, out_hbm, t_v, x_v, o_v):
    wid = lax.axis_index("s") * _NC + lax.axis_index("c")
    pltpu.sync_copy(t_hbm, t_v)
    pltpu.sync_copy(
        x_hbm.at[pl.ds(wid * _BPW * _SEQP, _BPW * _SEQP)], x_v
    )
    inv_l = jnp.float32(1.0 / _SEQ)
    lanes = lax.iota(jnp.int32, _L)
    z = jnp.zeros((_L,), jnp.float32)
    for g in range(_G):
        base = (g * _L + lanes) * _SEQP

        @plsc.parallel_loop(0, _SEQ, step=2, unroll=4, carry=(z, z, z, z))
        def body(l, accs, _base=base):
            a0, a1, b0, b1 = accs
            idx2 = plsc.load_gather(x_v, [_base + l]) * 2
            jdx2 = plsc.load_gather(x_v, [_base + (l + 1)]) * 2
            a0 = a0 + plsc.load_gather(t_v, [idx2])
            a1 = a1 + plsc.load_gather(t_v, [idx2 + 1])
            b0 = b0 + plsc.load_gather(t_v, [jdx2])
            b1 = b1 + plsc.load_gather(t_v, [jdx2 + 1])
            return a0, a1, b0, b1

        a0, a1, b0, b1 = body
        o_v[0, pl.ds(g * _L, _L)] = (a0 + b0) * inv_l
        o_v[1, pl.ds(g * _L, _L)] = (a1 + b1) * inv_l
    pltpu.sync_copy(o_v, out_hbm.at[wid])


_sc_pool = functools.partial(
    pl.kernel,
    out_type=jax.ShapeDtypeStruct((_NW, _OUT, _BPW), jnp.float32),
    mesh=plsc.VectorSubcoreMesh(
        core_axis_name="c", subcore_axis_name="s",
        num_cores=_NC, num_subcores=_NS,
    ),
    scratch_types=[
        pltpu.VMEM((_VOCAB_PAD * _OUT,), jnp.float32),
        pltpu.VMEM((_BPW * _SEQP,), jnp.int32),
        pltpu.VMEM((_OUT, _BPW), jnp.float32),
    ],
    compiler_params=pltpu.CompilerParams(needs_layout_passes=False),
)(_sc_body)


def kernel(x, attention_mask, emb_table, W, b):
    t = _project(emb_table, W, b).reshape(-1)
    x_p = jnp.pad(x, ((0, 0), (0, _SEQP - _SEQ))).reshape(-1)
    out = _sc_pool(t, x_p)
    return out.transpose(0, 2, 1).reshape(_BATCH, _OUT)


# speedup vs baseline: 1.5930x; 1.5068x over previous
import functools

import jax
import jax.numpy as jnp
from jax import lax
from jax.experimental import pallas as pl
from jax.experimental.pallas import tpu as pltpu
from jax.experimental.pallas import tpu_sc as plsc

_VOCAB_PAD = 30720
_BLK = 2048
_SEQ = 200
_BATCH = 4096
_OUT = 2
_NC, _NS, _L = 2, 16, 16
_NW = _NC * _NS
_BPW = _BATCH // _NW
_G = _BPW // _L


def _project_body(wt_ref, e_ref, b_ref, t_ref):
    t_ref[...] = (
        lax.dot_general(
            wt_ref[...], e_ref[...],
            dimension_numbers=(((1,), (1,)), ((), ())),
            preferred_element_type=jnp.float32,
        )
        + b_ref[...]
    )


def _project_t(emb_table, W, b):
    d = emb_table.shape[1]
    return pl.pallas_call(
        _project_body,
        grid=(_VOCAB_PAD // _BLK,),
        in_specs=[
            pl.BlockSpec((_OUT, d), lambda i: (0, 0)),
            pl.BlockSpec((_BLK, d), lambda i: (i, 0)),
            pl.BlockSpec((_OUT, 1), lambda i: (0, 0)),
        ],
        out_specs=pl.BlockSpec((_OUT, _BLK), lambda i: (0, i)),
        out_shape=jax.ShapeDtypeStruct((_OUT, _VOCAB_PAD), jnp.float32),
    )(W.T, emb_table, b.reshape(_OUT, 1))


def _sc_body(t_hbm, x_hbm, out_hbm, t_v, x_v, o_v):
    wid = lax.axis_index("s") * _NC + lax.axis_index("c")
    pltpu.sync_copy(t_hbm, t_v)
    pltpu.sync_copy(x_hbm.at[:, pl.ds(wid * _BPW, _BPW)], x_v)
    inv_l = jnp.float32(1.0 / _SEQ)
    z = jnp.zeros((_L,), jnp.float32)
    for g in range(_G):
        @plsc.parallel_loop(0, _SEQ, step=2, unroll=4, carry=(z, z, z, z))
        def body(l, accs, _g=g):
            a0, a1, b0, b1 = accs
            idx = x_v[l, pl.ds(_g * _L, _L)]
            jdx = x_v[l + 1, pl.ds(_g * _L, _L)]
            a0 = a0 + plsc.load_gather(t_v, [idx])
            a1 = a1 + plsc.load_gather(t_v, [idx + _VOCAB_PAD])
            b0 = b0 + plsc.load_gather(t_v, [jdx])
            b1 = b1 + plsc.load_gather(t_v, [jdx + _VOCAB_PAD])
            return a0, a1, b0, b1

        a0, a1, b0, b1 = body
        o_v[0, pl.ds(g * _L, _L)] = (a0 + b0) * inv_l
        o_v[1, pl.ds(g * _L, _L)] = (a1 + b1) * inv_l
    pltpu.sync_copy(o_v, out_hbm.at[wid])


_sc_pool = functools.partial(
    pl.kernel,
    out_type=jax.ShapeDtypeStruct((_NW, _OUT, _BPW), jnp.float32),
    mesh=plsc.VectorSubcoreMesh(
        core_axis_name="c", subcore_axis_name="s",
        num_cores=_NC, num_subcores=_NS,
    ),
    scratch_types=[
        pltpu.VMEM((_VOCAB_PAD * _OUT,), jnp.float32),
        pltpu.VMEM((_SEQ, _BPW), jnp.int32),
        pltpu.VMEM((_OUT, _BPW), jnp.float32),
    ],
    compiler_params=pltpu.CompilerParams(needs_layout_passes=False),
)(_sc_body)


def kernel(x, attention_mask, emb_table, W, b):
    t = _project_t(emb_table, W, b).reshape(-1)
    xt = x.T
    out = _sc_pool(t, xt)
    return out.transpose(0, 2, 1).reshape(_BATCH, _OUT)


# speedup vs baseline: 1.7688x; 1.1104x over previous
import functools

import jax
import jax.numpy as jnp
from jax import lax
from jax.experimental import pallas as pl
from jax.experimental.pallas import tpu as pltpu
from jax.experimental.pallas import tpu_sc as plsc

_VOCAB_PAD = 30720
_BLK = 2048
_SEQ = 200
_BATCH = 4096
_OUT = 2
_NC, _NS, _L = 2, 16, 16
_NW = _NC * _NS
_BPW = _BATCH // _NW
_G = _BPW // _L


def _project_body(wt_ref, e_ref, b_ref, t_ref):
    t2 = (
        lax.dot_general(
            wt_ref[...], e_ref[...],
            dimension_numbers=(((1,), (1,)), ((), ())),
            preferred_element_type=jnp.float32,
        )
        + b_ref[...]
    )
    u = lax.bitcast_convert_type(t2.astype(jnp.bfloat16), jnp.uint16)
    lo = u[0:1].astype(jnp.uint32)
    hi = u[1:2].astype(jnp.uint32) << 16
    t_ref[...] = lax.bitcast_convert_type(hi | lo, jnp.int32)


def _project_t(emb_table, W, b):
    d = emb_table.shape[1]
    return pl.pallas_call(
        _project_body,
        grid=(_VOCAB_PAD // _BLK,),
        in_specs=[
            pl.BlockSpec((_OUT, d), lambda i: (0, 0)),
            pl.BlockSpec((_BLK, d), lambda i: (i, 0)),
            pl.BlockSpec((_OUT, 1), lambda i: (0, 0)),
        ],
        out_specs=pl.BlockSpec((1, _BLK), lambda i: (0, i)),
        out_shape=jax.ShapeDtypeStruct((1, _VOCAB_PAD), jnp.int32),
    )(W.T, emb_table, b.reshape(_OUT, 1))


def _sc_body(t_hbm, x_hbm, out_hbm, t_v, x_v, o_v):
    wid = lax.axis_index("s") * _NC + lax.axis_index("c")
    pltpu.sync_copy(t_hbm, t_v)
    pltpu.sync_copy(x_hbm.at[:, pl.ds(wid * _BPW, _BPW)], x_v)
    inv_l = jnp.float32(1.0 / _SEQ)
    hmask = jnp.full((_L,), jnp.uint32(0xFFFF0000)).astype(jnp.int32)
    z = jnp.zeros((_L,), jnp.float32)
    for g in range(_G):
        @plsc.parallel_loop(0, _SEQ, step=2, unroll=4, carry=(z, z, z, z))
        def body(l, accs, _g=g):
            a0, a1, b0, b1 = accs
            pa = plsc.load_gather(t_v, [x_v[l, pl.ds(_g * _L, _L)]])
            pb = plsc.load_gather(t_v, [x_v[l + 1, pl.ds(_g * _L, _L)]])
            a0 = a0 + plsc.bitcast(pa << 16, jnp.float32)
            a1 = a1 + plsc.bitcast(pa & hmask, jnp.float32)
            b0 = b0 + plsc.bitcast(pb << 16, jnp.float32)
            b1 = b1 + plsc.bitcast(pb & hmask, jnp.float32)
            return a0, a1, b0, b1

        a0, a1, b0, b1 = body
        o_v[0, pl.ds(g * _L, _L)] = (a0 + b0) * inv_l
        o_v[1, pl.ds(g * _L, _L)] = (a1 + b1) * inv_l
    pltpu.sync_copy(o_v, out_hbm.at[wid])


_sc_pool = functools.partial(
    pl.kernel,
    out_type=jax.ShapeDtypeStruct((_NW, _OUT, _BPW), jnp.float32),
    mesh=plsc.VectorSubcoreMesh(
        core_axis_name="c", subcore_axis_name="s",
        num_cores=_NC, num_subcores=_NS,
    ),
    scratch_types=[
        pltpu.VMEM((_VOCAB_PAD,), jnp.int32),
        pltpu.VMEM((_SEQ, _BPW), jnp.int32),
        pltpu.VMEM((_OUT, _BPW), jnp.float32),
    ],
    compiler_params=pltpu.CompilerParams(needs_layout_passes=False),
)(_sc_body)


def kernel(x, attention_mask, emb_table, W, b):
    t = _project_t(emb_table, W, b).reshape(-1)
    xt = x.T
    out = _sc_pool(t, xt)
    return out.transpose(0, 2, 1).reshape(_BATCH, _OUT)


# speedup vs baseline: 1.7700x; 1.0007x over previous
import functools

import jax
import jax.numpy as jnp
from jax import lax
from jax.experimental import pallas as pl
from jax.experimental.pallas import tpu as pltpu
from jax.experimental.pallas import tpu_sc as plsc

_VOCAB_PAD = 30720
_BLK = 2048
_SEQ = 200
_BATCH = 4096
_OUT = 2
_NC, _NS, _L = 2, 16, 16
_NW = _NC * _NS
_BPW = _BATCH // _NW
_G = _BPW // _L


def _project_body(wt_ref, e_ref, b_ref, t_ref):
    t2 = (
        lax.dot_general(
            wt_ref[...], e_ref[...],
            dimension_numbers=(((1,), (1,)), ((), ())),
            preferred_element_type=jnp.float32,
        )
        + b_ref[...]
    )
    u = lax.bitcast_convert_type(t2.astype(jnp.bfloat16), jnp.uint16)
    lo = u[0:1].astype(jnp.uint32)
    hi = u[1:2].astype(jnp.uint32) << 16
    t_ref[...] = lax.bitcast_convert_type(hi | lo, jnp.int32)


def _project_t(emb_table, W, b):
    d = emb_table.shape[1]
    return pl.pallas_call(
        _project_body,
        grid=(_VOCAB_PAD // _BLK,),
        in_specs=[
            pl.BlockSpec((_OUT, d), lambda i: (0, 0)),
            pl.BlockSpec((_BLK, d), lambda i: (i, 0)),
            pl.BlockSpec((_OUT, 1), lambda i: (0, 0)),
        ],
        out_specs=pl.BlockSpec((1, _BLK), lambda i: (0, i)),
        out_shape=jax.ShapeDtypeStruct((1, _VOCAB_PAD), jnp.int32),
    )(W.T, emb_table, b.reshape(_OUT, 1))


def _sc_body(t_hbm, x_hbm, out_hbm, t_v, x_v, o_v):
    wid = lax.axis_index("s") * _NC + lax.axis_index("c")
    pltpu.sync_copy(t_hbm, t_v)
    pltpu.sync_copy(x_hbm.at[:, pl.ds(wid * _BPW, _BPW)], x_v)
    inv_l = jnp.float32(1.0 / _SEQ)
    hmask = jnp.full((_L,), jnp.uint32(0xFFFF0000)).astype(jnp.int32)
    z = jnp.zeros((_L,), jnp.float32)
    for g in range(_G):
        @plsc.parallel_loop(0, _SEQ, step=4, unroll=2,
                            carry=(z, z, z, z, z, z, z, z))
        def body(l, accs, _g=g):
            a0, a1, b0, b1, c0, c1, d0, d1 = accs
            pa = plsc.load_gather(t_v, [x_v[l, pl.ds(_g * _L, _L)]])
            pb = plsc.load_gather(t_v, [x_v[l + 1, pl.ds(_g * _L, _L)]])
            pc = plsc.load_gather(t_v, [x_v[l + 2, pl.ds(_g * _L, _L)]])
            pd = plsc.load_gather(t_v, [x_v[l + 3, pl.ds(_g * _L, _L)]])
            a0 = a0 + plsc.bitcast(pa << 16, jnp.float32)
            a1 = a1 + plsc.bitcast(pa & hmask, jnp.float32)
            b0 = b0 + plsc.bitcast(pb << 16, jnp.float32)
            b1 = b1 + plsc.bitcast(pb & hmask, jnp.float32)
            c0 = c0 + plsc.bitcast(pc << 16, jnp.float32)
            c1 = c1 + plsc.bitcast(pc & hmask, jnp.float32)
            d0 = d0 + plsc.bitcast(pd << 16, jnp.float32)
            d1 = d1 + plsc.bitcast(pd & hmask, jnp.float32)
            return a0, a1, b0, b1, c0, c1, d0, d1

        a0, a1, b0, b1, c0, c1, d0, d1 = body
        o_v[0, pl.ds(g * _L, _L)] = ((a0 + b0) + (c0 + d0)) * inv_l
        o_v[1, pl.ds(g * _L, _L)] = ((a1 + b1) + (c1 + d1)) * inv_l
    pltpu.sync_copy(o_v, out_hbm.at[wid])


_sc_pool = functools.partial(
    pl.kernel,
    out_type=jax.ShapeDtypeStruct((_NW, _OUT, _BPW), jnp.float32),
    mesh=plsc.VectorSubcoreMesh(
        core_axis_name="c", subcore_axis_name="s",
        num_cores=_NC, num_subcores=_NS,
    ),
    scratch_types=[
        pltpu.VMEM((_VOCAB_PAD,), jnp.int32),
        pltpu.VMEM((_SEQ, _BPW), jnp.int32),
        pltpu.VMEM((_OUT, _BPW), jnp.float32),
    ],
    compiler_params=pltpu.CompilerParams(needs_layout_passes=False),
)(_sc_body)


def kernel(x, attention_mask, emb_table, W, b):
    t = _project_t(emb_table, W, b).reshape(-1)
    xt = x.T
    out = _sc_pool(t, xt)
    return out.transpose(0, 2, 1).reshape(_BATCH, _OUT)
